# baseline (speedup 1.0000x reference)
import jax
import jax.numpy as jnp
from jax import lax
from jax.experimental import pallas as pl
from jax.experimental.pallas import tpu as pltpu
from jax.experimental.pallas import tpu_sc as plsc

B = 4096
L = 200
D = 32
NC = 2
NS = 16
NW = NC * NS
BPW = B // NW
RB = 8
NCHUNKS = BPW // RB
SUB = 40
KSUB = L // SUB
LANES = 16


def _body(x_hbm, tok_hbm, pos_hbm, out_hbm, idx_v, rows_v, pos_v, sem_g):
    wid = lax.axis_index("s") * NC + lax.axis_index("c")
    pltpu.sync_copy(pos_hbm, pos_v)

    @pl.loop(0, NCHUNKS)
    def _chunk(c):
        b0 = wid * BPW + c * RB

        pltpu.sync_copy(x_hbm.at[pl.ds(b0, RB)], idx_v)

        for r in range(RB):
            for j in range(KSUB):
                pltpu.async_copy(
                    tok_hbm.at[idx_v.at[r, pl.ds(j * SUB, SUB)]],
                    rows_v.at[r, pl.ds(j * SUB, SUB)],
                    sem_g,
                )

        pltpu.make_async_copy(out_hbm.at[pl.ds(0, RB)], rows_v, sem_g).wait()

        @pl.loop(0, L)
        def _mul(l):
            p0 = pos_v[l, pl.ds(0, LANES)]
            p1 = pos_v[l, pl.ds(LANES, LANES)]
            for r in range(RB):
                rows_v[r, l, pl.ds(0, LANES)] = rows_v[r, l, pl.ds(0, LANES)] * p0
                rows_v[r, l, pl.ds(LANES, LANES)] = (
                    rows_v[r, l, pl.ds(LANES, LANES)] * p1
                )

        pltpu.sync_copy(rows_v, out_hbm.at[pl.ds(b0, RB)])


@jax.jit
def _encode(x, token_table, pos_table):
    mesh = plsc.VectorSubcoreMesh(core_axis_name="c", subcore_axis_name="s")
    k = pl.kernel(
        _body,
        out_type=jax.ShapeDtypeStruct((B, L, D), jnp.float32),
        mesh=mesh,
        compiler_params=pltpu.CompilerParams(use_tc_tiling_on_sc=False),
        scratch_types=[
            pltpu.VMEM((RB, L), jnp.int32),
            pltpu.VMEM((RB, L, D), jnp.float32),
            pltpu.VMEM((L, D), jnp.float32),
            pltpu.SemaphoreType.DMA,
        ],
    )
    return k(x, token_table, pos_table)


def kernel(x, token_table, pos_table):
    return _encode(x.astype(jnp.int32), token_table, pos_table)
